# Initial kernel scaffold; baseline (speedup 1.0000x reference)
import jax
import jax.numpy as jnp
from jax.experimental import pallas as pl


def kernel(x, W1, b1, W2, b2):
    raise NotImplementedError("write your pallas kernel here")


# speedup vs baseline: 2.9889x; 2.9889x over previous
import functools

import jax
import jax.numpy as jnp
from jax.experimental import pallas as pl

TOKENS = 8192
INPUT_DIM = 4096
HIDDEN_DIM = 256
N_EXPERTS = 64

BLOCK = 512


def _gate_kernel(x_ref, w1_ref, b1_ref, w2_ref, b2_ref, gates_ref, idx_ref):
    x = x_ref[...]
    h = jax.lax.dot_general(
        x, w1_ref[...], (((1,), (1,)), ((), ())),
        preferred_element_type=jnp.float32)
    h = jnp.maximum(h + b1_ref[...], 0.0)
    logits = jax.lax.dot_general(
        h, w2_ref[...], (((1,), (1,)), ((), ())),
        preferred_element_type=jnp.float32)
    logits = logits + b2_ref[...]

    lanes = jax.lax.broadcasted_iota(jnp.int32, logits.shape, 1)
    l1 = jnp.max(logits, axis=-1, keepdims=True)
    i1 = jnp.argmax(logits, axis=-1).astype(jnp.int32)
    masked = jnp.where(lanes == i1[:, None], -jnp.inf, logits)
    l2 = jnp.max(masked, axis=-1, keepdims=True)
    i2 = jnp.argmax(masked, axis=-1).astype(jnp.int32)

    e = jnp.exp(l2 - l1)
    denom = 1.0 + e
    g1 = 1.0 / denom
    g2 = e / denom

    gates = jnp.where(lanes == i1[:, None], g1, 0.0)
    gates = jnp.where(lanes == i2[:, None], g2, gates)
    gates_ref[...] = gates
    idx_ref[...] = jnp.stack([i1, i2], axis=-1)


@jax.jit
def kernel(x, W1, b1, W2, b2):
    grid = (TOKENS // BLOCK,)
    gates, idx = pl.pallas_call(
        _gate_kernel,
        grid=grid,
        in_specs=[
            pl.BlockSpec((BLOCK, INPUT_DIM), lambda i: (i, 0)),
            pl.BlockSpec((HIDDEN_DIM, INPUT_DIM), lambda i: (0, 0)),
            pl.BlockSpec((1, HIDDEN_DIM), lambda i: (0, 0)),
            pl.BlockSpec((N_EXPERTS, HIDDEN_DIM), lambda i: (0, 0)),
            pl.BlockSpec((1, N_EXPERTS), lambda i: (0, 0)),
        ],
        out_specs=[
            pl.BlockSpec((BLOCK, N_EXPERTS), lambda i: (i, 0)),
            pl.BlockSpec((BLOCK, 2), lambda i: (i, 0)),
        ],
        out_shape=[
            jax.ShapeDtypeStruct((TOKENS, N_EXPERTS), jnp.float32),
            jax.ShapeDtypeStruct((TOKENS, 2), jnp.int32),
        ],
    )(x, W1, b1.reshape(1, HIDDEN_DIM), W2, b2.reshape(1, N_EXPERTS))
    return gates, idx


# speedup vs baseline: 3.0995x; 1.0370x over previous
import functools

import jax
import jax.numpy as jnp
from jax.experimental import pallas as pl

TOKENS = 8192
INPUT_DIM = 4096
HIDDEN_DIM = 256
N_EXPERTS = 64

BLOCK = 1024


def _gate_kernel(x_ref, w1_ref, b1_ref, w2_ref, b2_ref, gates_ref, idx_ref):
    x = x_ref[...]
    h = jax.lax.dot_general(
        x, w1_ref[...], (((1,), (1,)), ((), ())),
        preferred_element_type=jnp.float32)
    h = jnp.maximum(h + b1_ref[...], 0.0)
    logits = jax.lax.dot_general(
        h, w2_ref[...], (((1,), (1,)), ((), ())),
        preferred_element_type=jnp.float32)
    logits = logits + b2_ref[...]

    lanes = jax.lax.broadcasted_iota(jnp.int32, logits.shape, 1)
    l1 = jnp.max(logits, axis=-1, keepdims=True)
    i1 = jnp.argmax(logits, axis=-1).astype(jnp.int32)
    masked = jnp.where(lanes == i1[:, None], -jnp.inf, logits)
    l2 = jnp.max(masked, axis=-1, keepdims=True)
    i2 = jnp.argmax(masked, axis=-1).astype(jnp.int32)

    e = jnp.exp(l2 - l1)
    denom = 1.0 + e
    g1 = 1.0 / denom
    g2 = e / denom

    gates = jnp.where(lanes == i1[:, None], g1, 0.0)
    gates = jnp.where(lanes == i2[:, None], g2, gates)
    gates_ref[...] = gates
    idx_ref[...] = jnp.stack([i1, i2], axis=-1)


@jax.jit
def kernel(x, W1, b1, W2, b2):
    grid = (TOKENS // BLOCK,)
    gates, idx = pl.pallas_call(
        _gate_kernel,
        grid=grid,
        in_specs=[
            pl.BlockSpec((BLOCK, INPUT_DIM), lambda i: (i, 0)),
            pl.BlockSpec((HIDDEN_DIM, INPUT_DIM), lambda i: (0, 0)),
            pl.BlockSpec((1, HIDDEN_DIM), lambda i: (0, 0)),
            pl.BlockSpec((N_EXPERTS, HIDDEN_DIM), lambda i: (0, 0)),
            pl.BlockSpec((1, N_EXPERTS), lambda i: (0, 0)),
        ],
        out_specs=[
            pl.BlockSpec((BLOCK, N_EXPERTS), lambda i: (i, 0)),
            pl.BlockSpec((BLOCK, 2), lambda i: (i, 0)),
        ],
        out_shape=[
            jax.ShapeDtypeStruct((TOKENS, N_EXPERTS), jnp.float32),
            jax.ShapeDtypeStruct((TOKENS, 2), jnp.int32),
        ],
    )(x, W1, b1.reshape(1, HIDDEN_DIM), W2, b2.reshape(1, N_EXPERTS))
    return gates, idx
